# baseline (speedup 1.0000x reference)
import functools

import jax
import jax.numpy as jnp
from jax import lax
from jax.experimental import pallas as pl
from jax.experimental.pallas import tpu as pltpu
from jax.experimental.pallas import tpu_sc as plsc

N = 10000
E = 320000
D = 128
C = 64

NTILE = 16
NP = 10240
ROWS_PER_TILE = NP // NTILE
B = 128
EP = 2560 * B
EDGES_PER_TILE = EP // NTILE
NCHUNK = EDGES_PER_TILE // B


def _make_sc_agg(compute_deg):
    W = D + 16 if compute_deg else D
    mesh = plsc.VectorSubcoreMesh(core_axis_name="c", subcore_axis_name="s")
    if compute_deg:
        out_type = [jax.ShapeDtypeStruct((NP, D), jnp.float32),
                    jax.ShapeDtypeStruct((NP, D), jnp.float32),
                    jax.ShapeDtypeStruct((NP, 16), jnp.float32),
                    jax.ShapeDtypeStruct((NP, 16), jnp.float32)]
    else:
        out_type = [jax.ShapeDtypeStruct((NP, D), jnp.float32),
                    jax.ShapeDtypeStruct((NP, D), jnp.float32)]
    scratch_types = (
        [pltpu.VMEM((2, B), jnp.int32)] * 4 +
        [pltpu.VMEM((B, W), jnp.float32)] * 2 +
        [pltpu.VMEM_SHARED((NP, W), jnp.float32)] +
        [pltpu.SemaphoreType.DMA] * 8
    )

    @functools.partial(
        pl.kernel, mesh=mesh, out_type=out_type, scratch_types=scratch_types,
        compiler_params=pltpu.CompilerParams(use_tc_tiling_on_sc=False))
    def sc_agg(xa, xb, eda, edb, *refs):
        if compute_deg:
            agg_a, agg_b, deg_a, deg_b = refs[:4]
            rest = refs[4:]
        else:
            agg_a, agg_b = refs[:2]
            deg_a = deg_b = None
            rest = refs[2:]
        idx = list(rest[0:4])
        rows = list(rest[4:6])
        agg_sh = rest[6]
        isem = list(rest[7:11])
        gsem = list(rest[11:13])
        ssem = list(rest[13:15])
        rows0 = rows[0]

        c = lax.axis_index("c")
        s = lax.axis_index("s")
        row0 = s * ROWS_PER_TILE

        def zrow(j, _):
            r = j // (W // 16)
            k = j % (W // 16)
            rows0[r, pl.ds(k * 16, 16)] = jnp.zeros((16,), jnp.float32)
            return 0
        lax.fori_loop(0, (B * W) // 16, zrow, 0)

        for j in range(ROWS_PER_TILE // B):
            pltpu.sync_copy(rows0, agg_sh.at[pl.ds(row0 + j * B, B)])
        plsc.subcore_barrier()

        def run(x_hbm, ed_hbm):
            base = s * NCHUNK

            def step(i, u):
                m, m1, m2, m3 = u % 4, (u + 1) % 4, (u + 2) % 4, (u + 3) % 4
                p, q = u % 2, (u + 1) % 2
                @pl.when(i + 2 < NCHUNK)
                def _():
                    pltpu.async_copy(ed_hbm.at[base + i + 2], idx[m2], isem[m2])
                pltpu.make_async_copy(x_hbm.at[idx[m].at[0]], rows[p],
                                      gsem[p]).wait()
                @pl.when(i + 1 < NCHUNK)
                def _():
                    pltpu.make_async_copy(ed_hbm.at[base + i + 1], idx[m1],
                                          isem[m1]).wait()

                    @pl.when(i >= 1)
                    def _():
                        pltpu.make_async_copy(rows[q], agg_sh.at[idx[m3].at[1]],
                                              ssem[q]).wait()
                    pltpu.async_copy(x_hbm.at[idx[m1].at[0]], rows[q], gsem[q])
                pltpu.async_copy(rows[p], agg_sh.at[idx[m].at[1]], ssem[p],
                                 add=True)

            pltpu.sync_copy(ed_hbm.at[base], idx[0])
            pltpu.async_copy(ed_hbm.at[base + 1], idx[1], isem[1])
            pltpu.async_copy(x_hbm.at[idx[0].at[0]], rows[0], gsem[0])

            def body(g, _):
                for u in range(4):
                    step(4 * g + u, u)
                return 0
            lax.fori_loop(0, NCHUNK // 4, body, 0)
            for j in (NCHUNK - 2, NCHUNK - 1):
                pltpu.make_async_copy(rows[j % 2], agg_sh.at[idx[j % 4].at[1]],
                                      ssem[j % 2]).wait()

        @pl.when(c == 0)
        def _():
            run(xa, eda)

        @pl.when(c == 1)
        def _():
            run(xb, edb)

        plsc.subcore_barrier()

        def write_out(agg_out, deg_out):
            if compute_deg:
                pltpu.sync_copy(
                    agg_sh.at[pl.ds(row0, ROWS_PER_TILE), pl.ds(0, D)],
                    agg_out.at[pl.ds(row0, ROWS_PER_TILE)])
                pltpu.sync_copy(
                    agg_sh.at[pl.ds(row0, ROWS_PER_TILE), pl.ds(D, 16)],
                    deg_out.at[pl.ds(row0, ROWS_PER_TILE)])
            else:
                pltpu.sync_copy(agg_sh.at[pl.ds(row0, ROWS_PER_TILE)],
                                agg_out.at[pl.ds(row0, ROWS_PER_TILE)])

        @pl.when(c == 0)
        def _():
            write_out(agg_a, deg_a)

        @pl.when(c == 1)
        def _():
            write_out(agg_b, deg_b)

    return sc_agg


_sc_agg_deg = _make_sc_agg(compute_deg=True)
_sc_agg = _make_sc_agg(compute_deg=False)


BR = 1024


def _layer_body(x_ref, aa_ref, ab_ref, da_ref, db_ref,
                wsa_ref, wna_ref, wsb_ref, wnb_ref, oa_ref, ob_ref):
    xb = x_ref[...]
    ma = aa_ref[...] * (1.0 / jnp.maximum(da_ref[...], 1.0))
    mb = ab_ref[...] * (1.0 / jnp.maximum(db_ref[...], 1.0))
    oa_ref[...] = jnp.maximum(
        jnp.dot(xb, wsa_ref[...], preferred_element_type=jnp.float32)
        + jnp.dot(ma, wna_ref[...], preferred_element_type=jnp.float32), 0.0)
    ob_ref[...] = jnp.maximum(
        jnp.dot(xb, wsb_ref[...], preferred_element_type=jnp.float32)
        + jnp.dot(mb, wnb_ref[...], preferred_element_type=jnp.float32), 0.0)


def _tc_layer1(x, agg_a, agg_b, deg_a, deg_b, wsa, wna, wsb, wnb):
    row_spec = pl.BlockSpec((BR, D), lambda i: (i, 0))
    deg_spec = pl.BlockSpec((BR, 1), lambda i: (i, 0))
    w_spec = pl.BlockSpec((D, D), lambda i: (0, 0))
    return pl.pallas_call(
        _layer_body,
        grid=(NP // BR,),
        in_specs=[row_spec, row_spec, row_spec, deg_spec, deg_spec,
                  w_spec, w_spec, w_spec, w_spec],
        out_specs=[row_spec, row_spec],
        out_shape=[jax.ShapeDtypeStruct((NP, D), jnp.float32),
                   jax.ShapeDtypeStruct((NP, D), jnp.float32)],
    )(x, agg_a, agg_b, deg_a, deg_b, wsa, wna, wsb, wnb)


def _final_body(ha_ref, hb_ref, aa_ref, ab_ref, da_ref, db_ref,
                wsa_ref, wna_ref, wsb_ref, wnb_ref, wla_ref, wlb_ref,
                out_ref, acc_a, acc_b):
    i = pl.program_id(0)
    ma = aa_ref[...] * (1.0 / jnp.maximum(da_ref[...], 1.0))
    mb = ab_ref[...] * (1.0 / jnp.maximum(db_ref[...], 1.0))
    h2a = jnp.maximum(
        jnp.dot(ha_ref[...], wsa_ref[...], preferred_element_type=jnp.float32)
        + jnp.dot(ma, wna_ref[...], preferred_element_type=jnp.float32), 0.0)
    h2b = jnp.maximum(
        jnp.dot(hb_ref[...], wsb_ref[...], preferred_element_type=jnp.float32)
        + jnp.dot(mb, wnb_ref[...], preferred_element_type=jnp.float32), 0.0)
    rid = i * BR + lax.broadcasted_iota(jnp.int32, (BR, D), 0)
    mask = (rid < N).astype(jnp.float32)
    ca = jnp.sum(h2a * mask, axis=0, keepdims=True)
    cb = jnp.sum(h2b * mask, axis=0, keepdims=True)

    @pl.when(i == 0)
    def _():
        acc_a[...] = ca
        acc_b[...] = cb

    @pl.when(i > 0)
    def _():
        acc_a[...] += ca
        acc_b[...] += cb

    @pl.when(i == (NP // BR) - 1)
    def _():
        ra = jnp.dot(acc_a[...] * (1.0 / N), wla_ref[...],
                     preferred_element_type=jnp.float32)
        rb = jnp.dot(acc_b[...] * (1.0 / N), wlb_ref[...],
                     preferred_element_type=jnp.float32)
        sa = 1.0 / (1.0 + jnp.exp(-ra))
        sb = 1.0 / (1.0 + jnp.exp(-rb))
        out_ref[...] = (sa + sb) * 0.5


def _tc_final(h1a, h1b, agg_a, agg_b, deg_a, deg_b, wsa, wna, wsb, wnb, wla, wlb):
    row_spec = pl.BlockSpec((BR, D), lambda i: (i, 0))
    deg_spec = pl.BlockSpec((BR, 1), lambda i: (i, 0))
    w_spec = pl.BlockSpec((D, D), lambda i: (0, 0))
    wl_spec = pl.BlockSpec((D, C), lambda i: (0, 0))
    return pl.pallas_call(
        _final_body,
        grid=(NP // BR,),
        in_specs=[row_spec, row_spec, row_spec, row_spec, deg_spec, deg_spec,
                  w_spec, w_spec, w_spec, w_spec, wl_spec, wl_spec],
        out_specs=pl.BlockSpec((1, C), lambda i: (0, 0)),
        out_shape=jax.ShapeDtypeStruct((1, C), jnp.float32),
        scratch_shapes=[pltpu.VMEM((1, D), jnp.float32),
                        pltpu.VMEM((1, D), jnp.float32)],
    )(h1a, h1b, agg_a, agg_b, deg_a, deg_b, wsa, wna, wsb, wnb, wla, wlb)


def kernel(H1_edge_index, H2_edge_index, feats,
           W1s_a, W1n_a, W2s_a, W2n_a, Wl_a,
           W1s_b, W1n_b, W2s_b, W2n_b, Wl_b):
    pad_src = jnp.zeros((EP - E,), jnp.int32)
    pad_dst = jnp.full((EP - E,), N, jnp.int32)
    eda = jnp.stack(
        [jnp.concatenate([H1_edge_index[0], pad_src]).reshape(EP // B, B),
         jnp.concatenate([H1_edge_index[1], pad_dst]).reshape(EP // B, B)],
        axis=1)
    edb = jnp.stack(
        [jnp.concatenate([H2_edge_index[0], pad_src]).reshape(EP // B, B),
         jnp.concatenate([H2_edge_index[1], pad_dst]).reshape(EP // B, B)],
        axis=1)
    x = jnp.pad(feats, ((0, NP - N), (0, 0)))
    xw = jnp.concatenate(
        [x, jnp.ones((NP, 1), jnp.float32), jnp.zeros((NP, 15), jnp.float32)],
        axis=1)

    agg1_a, agg1_b, degc_a, degc_b = _sc_agg_deg(xw, xw, eda, edb)
    deg_a = degc_a[:, 0:1]
    deg_b = degc_b[:, 0:1]
    h1a, h1b = _tc_layer1(x, agg1_a, agg1_b, deg_a, deg_b,
                          W1s_a, W1n_a, W1s_b, W1n_b)
    agg2_a, agg2_b = _sc_agg(h1a, h1b, eda, edb)
    return _tc_final(h1a, h1b, agg2_a, agg2_b, deg_a, deg_b,
                     W2s_a, W2n_a, W2s_b, W2n_b, Wl_a, Wl_b)


# speedup vs baseline: 1.0258x; 1.0258x over previous
import functools

import jax
import jax.numpy as jnp
from jax import lax
from jax.experimental import pallas as pl
from jax.experimental.pallas import tpu as pltpu
from jax.experimental.pallas import tpu_sc as plsc

N = 10000
E = 320000
D = 128
C = 64

NTILE = 16
NP = 10240
ROWS_PER_TILE = NP // NTILE
B = 128
EP = 2560 * B
EDGES_PER_TILE = EP // NTILE
NCHUNK = EDGES_PER_TILE // B


def _make_sc_agg(compute_deg):
    W = D + 16 if compute_deg else D
    mesh = plsc.VectorSubcoreMesh(core_axis_name="c", subcore_axis_name="s")
    if compute_deg:
        out_type = [jax.ShapeDtypeStruct((NP, D), jnp.float32),
                    jax.ShapeDtypeStruct((NP, D), jnp.float32),
                    jax.ShapeDtypeStruct((NP, 16), jnp.float32),
                    jax.ShapeDtypeStruct((NP, 16), jnp.float32)]
    else:
        out_type = [jax.ShapeDtypeStruct((NP, D), jnp.float32),
                    jax.ShapeDtypeStruct((NP, D), jnp.float32)]
    scratch_types = (
        [pltpu.VMEM((2, B), jnp.int32)] * 4 +
        [pltpu.VMEM((B, W), jnp.float32)] * 2 +
        [pltpu.VMEM_SHARED((NP, W), jnp.float32)] +
        [pltpu.SemaphoreType.DMA] * 8
    )

    kernel_kwargs = dict(mesh=mesh, out_type=out_type,
                         scratch_types=scratch_types)
    if compute_deg:
        kernel_kwargs["compiler_params"] = pltpu.CompilerParams(
            use_tc_tiling_on_sc=False)

    @functools.partial(pl.kernel, **kernel_kwargs)
    def sc_agg(xa, xb, eda, edb, *refs):
        if compute_deg:
            agg_a, agg_b, deg_a, deg_b = refs[:4]
            rest = refs[4:]
        else:
            agg_a, agg_b = refs[:2]
            deg_a = deg_b = None
            rest = refs[2:]
        idx = list(rest[0:4])
        rows = list(rest[4:6])
        agg_sh = rest[6]
        isem = list(rest[7:11])
        gsem = list(rest[11:13])
        ssem = list(rest[13:15])
        rows0 = rows[0]

        c = lax.axis_index("c")
        s = lax.axis_index("s")
        row0 = s * ROWS_PER_TILE

        def zrow(j, _):
            r = j // (W // 16)
            k = j % (W // 16)
            rows0[r, pl.ds(k * 16, 16)] = jnp.zeros((16,), jnp.float32)
            return 0
        lax.fori_loop(0, (B * W) // 16, zrow, 0)

        for j in range(ROWS_PER_TILE // B):
            pltpu.sync_copy(rows0, agg_sh.at[pl.ds(row0 + j * B, B)])
        plsc.subcore_barrier()

        def run(x_hbm, ed_hbm):
            base = s * NCHUNK

            def step(i, u):
                m, m1, m2, m3 = u % 4, (u + 1) % 4, (u + 2) % 4, (u + 3) % 4
                p, q = u % 2, (u + 1) % 2
                @pl.when(i + 2 < NCHUNK)
                def _():
                    pltpu.async_copy(ed_hbm.at[base + i + 2], idx[m2], isem[m2])
                pltpu.make_async_copy(x_hbm.at[idx[m].at[0]], rows[p],
                                      gsem[p]).wait()
                @pl.when(i + 1 < NCHUNK)
                def _():
                    pltpu.make_async_copy(ed_hbm.at[base + i + 1], idx[m1],
                                          isem[m1]).wait()

                    @pl.when(i >= 1)
                    def _():
                        pltpu.make_async_copy(rows[q], agg_sh.at[idx[m3].at[1]],
                                              ssem[q]).wait()
                    pltpu.async_copy(x_hbm.at[idx[m1].at[0]], rows[q], gsem[q])
                pltpu.async_copy(rows[p], agg_sh.at[idx[m].at[1]], ssem[p],
                                 add=True)

            pltpu.sync_copy(ed_hbm.at[base], idx[0])
            pltpu.async_copy(ed_hbm.at[base + 1], idx[1], isem[1])
            pltpu.async_copy(x_hbm.at[idx[0].at[0]], rows[0], gsem[0])

            def body(g, _):
                for u in range(4):
                    step(4 * g + u, u)
                return 0
            lax.fori_loop(0, NCHUNK // 4, body, 0)
            for j in (NCHUNK - 2, NCHUNK - 1):
                pltpu.make_async_copy(rows[j % 2], agg_sh.at[idx[j % 4].at[1]],
                                      ssem[j % 2]).wait()

        @pl.when(c == 0)
        def _():
            run(xa, eda)

        @pl.when(c == 1)
        def _():
            run(xb, edb)

        plsc.subcore_barrier()

        def write_out(agg_out, deg_out):
            if compute_deg:
                pltpu.sync_copy(
                    agg_sh.at[pl.ds(row0, ROWS_PER_TILE), pl.ds(0, D)],
                    agg_out.at[pl.ds(row0, ROWS_PER_TILE)])
                pltpu.sync_copy(
                    agg_sh.at[pl.ds(row0, ROWS_PER_TILE), pl.ds(D, 16)],
                    deg_out.at[pl.ds(row0, ROWS_PER_TILE)])
            else:
                pltpu.sync_copy(agg_sh.at[pl.ds(row0, ROWS_PER_TILE)],
                                agg_out.at[pl.ds(row0, ROWS_PER_TILE)])

        @pl.when(c == 0)
        def _():
            write_out(agg_a, deg_a)

        @pl.when(c == 1)
        def _():
            write_out(agg_b, deg_b)

    return sc_agg


_sc_agg_deg = _make_sc_agg(compute_deg=True)
_sc_agg = _make_sc_agg(compute_deg=False)


BR = 1024


def _layer_body(x_ref, aa_ref, ab_ref, da_ref, db_ref,
                wsa_ref, wna_ref, wsb_ref, wnb_ref, oa_ref, ob_ref):
    xb = x_ref[...]
    ma = aa_ref[...] * (1.0 / jnp.maximum(da_ref[...], 1.0))
    mb = ab_ref[...] * (1.0 / jnp.maximum(db_ref[...], 1.0))
    oa_ref[...] = jnp.maximum(
        jnp.dot(xb, wsa_ref[...], preferred_element_type=jnp.float32)
        + jnp.dot(ma, wna_ref[...], preferred_element_type=jnp.float32), 0.0)
    ob_ref[...] = jnp.maximum(
        jnp.dot(xb, wsb_ref[...], preferred_element_type=jnp.float32)
        + jnp.dot(mb, wnb_ref[...], preferred_element_type=jnp.float32), 0.0)


def _tc_layer1(x, agg_a, agg_b, deg_a, deg_b, wsa, wna, wsb, wnb):
    row_spec = pl.BlockSpec((BR, D), lambda i: (i, 0))
    deg_spec = pl.BlockSpec((BR, 1), lambda i: (i, 0))
    w_spec = pl.BlockSpec((D, D), lambda i: (0, 0))
    return pl.pallas_call(
        _layer_body,
        grid=(NP // BR,),
        in_specs=[row_spec, row_spec, row_spec, deg_spec, deg_spec,
                  w_spec, w_spec, w_spec, w_spec],
        out_specs=[row_spec, row_spec],
        out_shape=[jax.ShapeDtypeStruct((NP, D), jnp.float32),
                   jax.ShapeDtypeStruct((NP, D), jnp.float32)],
    )(x, agg_a, agg_b, deg_a, deg_b, wsa, wna, wsb, wnb)


def _final_body(ha_ref, hb_ref, aa_ref, ab_ref, da_ref, db_ref,
                wsa_ref, wna_ref, wsb_ref, wnb_ref, wla_ref, wlb_ref,
                out_ref, acc_a, acc_b):
    i = pl.program_id(0)
    ma = aa_ref[...] * (1.0 / jnp.maximum(da_ref[...], 1.0))
    mb = ab_ref[...] * (1.0 / jnp.maximum(db_ref[...], 1.0))
    h2a = jnp.maximum(
        jnp.dot(ha_ref[...], wsa_ref[...], preferred_element_type=jnp.float32)
        + jnp.dot(ma, wna_ref[...], preferred_element_type=jnp.float32), 0.0)
    h2b = jnp.maximum(
        jnp.dot(hb_ref[...], wsb_ref[...], preferred_element_type=jnp.float32)
        + jnp.dot(mb, wnb_ref[...], preferred_element_type=jnp.float32), 0.0)
    rid = i * BR + lax.broadcasted_iota(jnp.int32, (BR, D), 0)
    mask = (rid < N).astype(jnp.float32)
    ca = jnp.sum(h2a * mask, axis=0, keepdims=True)
    cb = jnp.sum(h2b * mask, axis=0, keepdims=True)

    @pl.when(i == 0)
    def _():
        acc_a[...] = ca
        acc_b[...] = cb

    @pl.when(i > 0)
    def _():
        acc_a[...] += ca
        acc_b[...] += cb

    @pl.when(i == (NP // BR) - 1)
    def _():
        ra = jnp.dot(acc_a[...] * (1.0 / N), wla_ref[...],
                     preferred_element_type=jnp.float32)
        rb = jnp.dot(acc_b[...] * (1.0 / N), wlb_ref[...],
                     preferred_element_type=jnp.float32)
        sa = 1.0 / (1.0 + jnp.exp(-ra))
        sb = 1.0 / (1.0 + jnp.exp(-rb))
        out_ref[...] = (sa + sb) * 0.5


def _tc_final(h1a, h1b, agg_a, agg_b, deg_a, deg_b, wsa, wna, wsb, wnb, wla, wlb):
    row_spec = pl.BlockSpec((BR, D), lambda i: (i, 0))
    deg_spec = pl.BlockSpec((BR, 1), lambda i: (i, 0))
    w_spec = pl.BlockSpec((D, D), lambda i: (0, 0))
    wl_spec = pl.BlockSpec((D, C), lambda i: (0, 0))
    return pl.pallas_call(
        _final_body,
        grid=(NP // BR,),
        in_specs=[row_spec, row_spec, row_spec, row_spec, deg_spec, deg_spec,
                  w_spec, w_spec, w_spec, w_spec, wl_spec, wl_spec],
        out_specs=pl.BlockSpec((1, C), lambda i: (0, 0)),
        out_shape=jax.ShapeDtypeStruct((1, C), jnp.float32),
        scratch_shapes=[pltpu.VMEM((1, D), jnp.float32),
                        pltpu.VMEM((1, D), jnp.float32)],
    )(h1a, h1b, agg_a, agg_b, deg_a, deg_b, wsa, wna, wsb, wnb, wla, wlb)


def kernel(H1_edge_index, H2_edge_index, feats,
           W1s_a, W1n_a, W2s_a, W2n_a, Wl_a,
           W1s_b, W1n_b, W2s_b, W2n_b, Wl_b):
    pad_src = jnp.zeros((EP - E,), jnp.int32)
    pad_dst = jnp.full((EP - E,), N, jnp.int32)
    eda = jnp.stack(
        [jnp.concatenate([H1_edge_index[0], pad_src]).reshape(EP // B, B),
         jnp.concatenate([H1_edge_index[1], pad_dst]).reshape(EP // B, B)],
        axis=1)
    edb = jnp.stack(
        [jnp.concatenate([H2_edge_index[0], pad_src]).reshape(EP // B, B),
         jnp.concatenate([H2_edge_index[1], pad_dst]).reshape(EP // B, B)],
        axis=1)
    x = jnp.pad(feats, ((0, NP - N), (0, 0)))
    xw = jnp.concatenate(
        [x, jnp.ones((NP, 1), jnp.float32), jnp.zeros((NP, 15), jnp.float32)],
        axis=1)

    agg1_a, agg1_b, degc_a, degc_b = _sc_agg_deg(xw, xw, eda, edb)
    deg_a = degc_a[:, 0:1]
    deg_b = degc_b[:, 0:1]
    h1a, h1b = _tc_layer1(x, agg1_a, agg1_b, deg_a, deg_b,
                          W1s_a, W1n_a, W1s_b, W1n_b)
    agg2_a, agg2_b = _sc_agg(h1a, h1b, eda, edb)
    return _tc_final(h1a, h1b, agg2_a, agg2_b, deg_a, deg_b,
                     W2s_a, W2n_a, W2s_b, W2n_b, Wl_a, Wl_b)


# speedup vs baseline: 1.6594x; 1.6177x over previous
import functools

import jax
import jax.numpy as jnp
from jax import lax
from jax.experimental import pallas as pl
from jax.experimental.pallas import tpu as pltpu
from jax.experimental.pallas import tpu_sc as plsc

N = 10000
E = 320000
D = 128
C = 64

NTILE = 16
NP = 10240
ROWS_PER_TILE = NP // NTILE
B = 128
EP = 2560 * B
EDGES_PER_TILE = EP // NTILE
NCHUNK = EDGES_PER_TILE // B


def _make_sc_agg(compute_deg):
    mesh = plsc.VectorSubcoreMesh(core_axis_name="c", subcore_axis_name="s")
    out_type = [jax.ShapeDtypeStruct((NP, D), jnp.bfloat16),
                jax.ShapeDtypeStruct((NP, D), jnp.bfloat16)]
    if compute_deg:
        out_type += [jax.ShapeDtypeStruct((NP,), jnp.float32),
                     jax.ShapeDtypeStruct((NP,), jnp.float32)]
    scratch_types = (
        [pltpu.VMEM((2, B), jnp.int32)] * 4 +
        [pltpu.VMEM((B, D), jnp.bfloat16)] * 2 +
        [pltpu.VMEM((B,), jnp.float32),
         pltpu.VMEM((ROWS_PER_TILE,), jnp.float32),
         pltpu.VMEM_SHARED((NP, D), jnp.bfloat16),
         pltpu.VMEM_SHARED((NP,), jnp.float32)] +
        [pltpu.SemaphoreType.DMA] * 10
    )

    @functools.partial(
        pl.kernel, mesh=mesh, out_type=out_type, scratch_types=scratch_types,
        compiler_params=pltpu.CompilerParams(use_tc_tiling_on_sc=False))
    def sc_agg(xa, xb, eda, edb, *refs):
        if compute_deg:
            agg_a, agg_b, deg_a, deg_b = refs[:4]
            rest = refs[4:]
        else:
            agg_a, agg_b = refs[:2]
            deg_a = deg_b = None
            rest = refs[2:]
        idx = list(rest[0:4])
        rows = list(rest[4:6])
        ones_v, zeros_v, agg_sh, deg_sh = rest[6:10]
        isem = list(rest[10:14])
        gsem = list(rest[14:16])
        ssem = list(rest[16:18])
        dsem = list(rest[18:20])
        rows0 = rows[0]

        c = lax.axis_index("c")
        s = lax.axis_index("s")
        row0 = s * ROWS_PER_TILE

        def zrow(j, _):
            r = j // (D // 32)
            k = j % (D // 32)
            rows0[r, pl.ds(k * 32, 32)] = jnp.zeros((32,), jnp.bfloat16)
            return 0
        lax.fori_loop(0, (B * D) // 32, zrow, 0)

        def zvec(j, _):
            zeros_v[pl.ds(j * 16, 16)] = jnp.zeros((16,), jnp.float32)
            return 0
        lax.fori_loop(0, ROWS_PER_TILE // 16, zvec, 0)
        if compute_deg:
            def ovec(j, _):
                ones_v[pl.ds(j * 16, 16)] = jnp.ones((16,), jnp.float32)
                return 0
            lax.fori_loop(0, B // 16, ovec, 0)

        for j in range(ROWS_PER_TILE // B):
            pltpu.sync_copy(rows0, agg_sh.at[pl.ds(row0 + j * B, B)])
        if compute_deg:
            pltpu.sync_copy(zeros_v, deg_sh.at[pl.ds(row0, ROWS_PER_TILE)])
        plsc.subcore_barrier()

        def run(x_hbm, ed_hbm):
            base = s * NCHUNK

            def step(i, u):
                m, m1, m2, m3 = u % 4, (u + 1) % 4, (u + 2) % 4, (u + 3) % 4
                p, q = u % 2, (u + 1) % 2
                @pl.when(i + 2 < NCHUNK)
                def _():
                    pltpu.async_copy(ed_hbm.at[base + i + 2], idx[m2], isem[m2])
                pltpu.make_async_copy(x_hbm.at[idx[m].at[0]], rows[p],
                                      gsem[p]).wait()
                @pl.when(i + 1 < NCHUNK)
                def _():
                    pltpu.make_async_copy(ed_hbm.at[base + i + 1], idx[m1],
                                          isem[m1]).wait()

                    @pl.when(i >= 1)
                    def _():
                        pltpu.make_async_copy(rows[q], agg_sh.at[idx[m3].at[1]],
                                              ssem[q]).wait()
                        if compute_deg:
                            pltpu.make_async_copy(ones_v,
                                                  deg_sh.at[idx[m3].at[1]],
                                                  dsem[q]).wait()
                    pltpu.async_copy(x_hbm.at[idx[m1].at[0]], rows[q], gsem[q])
                pltpu.async_copy(rows[p], agg_sh.at[idx[m].at[1]], ssem[p],
                                 add=True)
                if compute_deg:
                    pltpu.async_copy(ones_v, deg_sh.at[idx[m].at[1]], dsem[p],
                                     add=True)

            pltpu.sync_copy(ed_hbm.at[base], idx[0])
            pltpu.async_copy(ed_hbm.at[base + 1], idx[1], isem[1])
            pltpu.async_copy(x_hbm.at[idx[0].at[0]], rows[0], gsem[0])

            def body(g, _):
                for u in range(4):
                    step(4 * g + u, u)
                return 0
            lax.fori_loop(0, NCHUNK // 4, body, 0)
            for j in (NCHUNK - 2, NCHUNK - 1):
                pltpu.make_async_copy(rows[j % 2], agg_sh.at[idx[j % 4].at[1]],
                                      ssem[j % 2]).wait()
                if compute_deg:
                    pltpu.make_async_copy(ones_v, deg_sh.at[idx[j % 4].at[1]],
                                          dsem[j % 2]).wait()

        @pl.when(c == 0)
        def _():
            run(xa, eda)

        @pl.when(c == 1)
        def _():
            run(xb, edb)

        plsc.subcore_barrier()

        def write_out(agg_out, deg_out):
            pltpu.sync_copy(agg_sh.at[pl.ds(row0, ROWS_PER_TILE)],
                            agg_out.at[pl.ds(row0, ROWS_PER_TILE)])
            if compute_deg:
                pltpu.sync_copy(deg_sh.at[pl.ds(row0, ROWS_PER_TILE)],
                                deg_out.at[pl.ds(row0, ROWS_PER_TILE)])

        @pl.when(c == 0)
        def _():
            write_out(agg_a, deg_a)

        @pl.when(c == 1)
        def _():
            write_out(agg_b, deg_b)

    return sc_agg


_sc_agg_deg = _make_sc_agg(compute_deg=True)
_sc_agg = _make_sc_agg(compute_deg=False)


BR = 1024


def _layer_body(x_ref, aa_ref, ab_ref, da_ref, db_ref,
                wsa_ref, wna_ref, wsb_ref, wnb_ref, oa_ref, ob_ref):
    xb = x_ref[...]
    ma = aa_ref[...].astype(jnp.float32) * (1.0 / jnp.maximum(da_ref[...], 1.0))
    mb = ab_ref[...].astype(jnp.float32) * (1.0 / jnp.maximum(db_ref[...], 1.0))
    oa_ref[...] = jnp.maximum(
        jnp.dot(xb, wsa_ref[...], preferred_element_type=jnp.float32)
        + jnp.dot(ma, wna_ref[...], preferred_element_type=jnp.float32), 0.0)
    ob_ref[...] = jnp.maximum(
        jnp.dot(xb, wsb_ref[...], preferred_element_type=jnp.float32)
        + jnp.dot(mb, wnb_ref[...], preferred_element_type=jnp.float32), 0.0)


def _tc_layer1(x, agg_a, agg_b, deg_a, deg_b, wsa, wna, wsb, wnb):
    row_spec = pl.BlockSpec((BR, D), lambda i: (i, 0))
    deg_spec = pl.BlockSpec((BR, 1), lambda i: (i, 0))
    w_spec = pl.BlockSpec((D, D), lambda i: (0, 0))
    return pl.pallas_call(
        _layer_body,
        grid=(NP // BR,),
        in_specs=[row_spec, row_spec, row_spec, deg_spec, deg_spec,
                  w_spec, w_spec, w_spec, w_spec],
        out_specs=[row_spec, row_spec],
        out_shape=[jax.ShapeDtypeStruct((NP, D), jnp.float32),
                   jax.ShapeDtypeStruct((NP, D), jnp.float32)],
    )(x, agg_a, agg_b, deg_a, deg_b, wsa, wna, wsb, wnb)


def _final_body(ha_ref, hb_ref, aa_ref, ab_ref, da_ref, db_ref,
                wsa_ref, wna_ref, wsb_ref, wnb_ref, wla_ref, wlb_ref,
                out_ref, acc_a, acc_b):
    i = pl.program_id(0)
    ma = aa_ref[...].astype(jnp.float32) * (1.0 / jnp.maximum(da_ref[...], 1.0))
    mb = ab_ref[...].astype(jnp.float32) * (1.0 / jnp.maximum(db_ref[...], 1.0))
    h2a = jnp.maximum(
        jnp.dot(ha_ref[...], wsa_ref[...], preferred_element_type=jnp.float32)
        + jnp.dot(ma, wna_ref[...], preferred_element_type=jnp.float32), 0.0)
    h2b = jnp.maximum(
        jnp.dot(hb_ref[...], wsb_ref[...], preferred_element_type=jnp.float32)
        + jnp.dot(mb, wnb_ref[...], preferred_element_type=jnp.float32), 0.0)
    rid = i * BR + lax.broadcasted_iota(jnp.int32, (BR, D), 0)
    mask = (rid < N).astype(jnp.float32)
    ca = jnp.sum(h2a * mask, axis=0, keepdims=True)
    cb = jnp.sum(h2b * mask, axis=0, keepdims=True)

    @pl.when(i == 0)
    def _():
        acc_a[...] = ca
        acc_b[...] = cb

    @pl.when(i > 0)
    def _():
        acc_a[...] += ca
        acc_b[...] += cb

    @pl.when(i == (NP // BR) - 1)
    def _():
        ra = jnp.dot(acc_a[...] * (1.0 / N), wla_ref[...],
                     preferred_element_type=jnp.float32)
        rb = jnp.dot(acc_b[...] * (1.0 / N), wlb_ref[...],
                     preferred_element_type=jnp.float32)
        sa = 1.0 / (1.0 + jnp.exp(-ra))
        sb = 1.0 / (1.0 + jnp.exp(-rb))
        out_ref[...] = (sa + sb) * 0.5


def _tc_final(h1a, h1b, agg_a, agg_b, deg_a, deg_b, wsa, wna, wsb, wnb, wla, wlb):
    row_spec = pl.BlockSpec((BR, D), lambda i: (i, 0))
    deg_spec = pl.BlockSpec((BR, 1), lambda i: (i, 0))
    w_spec = pl.BlockSpec((D, D), lambda i: (0, 0))
    wl_spec = pl.BlockSpec((D, C), lambda i: (0, 0))
    return pl.pallas_call(
        _final_body,
        grid=(NP // BR,),
        in_specs=[row_spec, row_spec, row_spec, row_spec, deg_spec, deg_spec,
                  w_spec, w_spec, w_spec, w_spec, wl_spec, wl_spec],
        out_specs=pl.BlockSpec((1, C), lambda i: (0, 0)),
        out_shape=jax.ShapeDtypeStruct((1, C), jnp.float32),
        scratch_shapes=[pltpu.VMEM((1, D), jnp.float32),
                        pltpu.VMEM((1, D), jnp.float32)],
    )(h1a, h1b, agg_a, agg_b, deg_a, deg_b, wsa, wna, wsb, wnb, wla, wlb)


def kernel(H1_edge_index, H2_edge_index, feats,
           W1s_a, W1n_a, W2s_a, W2n_a, Wl_a,
           W1s_b, W1n_b, W2s_b, W2n_b, Wl_b):
    pad_src = jnp.zeros((EP - E,), jnp.int32)
    pad_dst = jnp.full((EP - E,), N, jnp.int32)
    eda = jnp.stack(
        [jnp.concatenate([H1_edge_index[0], pad_src]).reshape(EP // B, B),
         jnp.concatenate([H1_edge_index[1], pad_dst]).reshape(EP // B, B)],
        axis=1)
    edb = jnp.stack(
        [jnp.concatenate([H2_edge_index[0], pad_src]).reshape(EP // B, B),
         jnp.concatenate([H2_edge_index[1], pad_dst]).reshape(EP // B, B)],
        axis=1)
    x = jnp.pad(feats, ((0, NP - N), (0, 0)))
    xbf = x.astype(jnp.bfloat16)

    agg1_a, agg1_b, deg_a, deg_b = _sc_agg_deg(xbf, xbf, eda, edb)
    deg_a = deg_a.reshape(NP, 1)
    deg_b = deg_b.reshape(NP, 1)
    h1a, h1b = _tc_layer1(x, agg1_a, agg1_b, deg_a, deg_b,
                          W1s_a, W1n_a, W1s_b, W1n_b)
    agg2_a, agg2_b = _sc_agg(h1a.astype(jnp.bfloat16),
                             h1b.astype(jnp.bfloat16), eda, edb)
    return _tc_final(h1a, h1b, agg2_a, agg2_b, deg_a, deg_b,
                     W2s_a, W2n_a, W2s_b, W2n_b, Wl_a, Wl_b)


# speedup vs baseline: 1.6660x; 1.0040x over previous
import functools

import jax
import jax.numpy as jnp
from jax import lax
from jax.experimental import pallas as pl
from jax.experimental.pallas import tpu as pltpu
from jax.experimental.pallas import tpu_sc as plsc

N = 10000
E = 320000
D = 128
C = 64

NTILE = 16
NP = 10240
ROWS_PER_TILE = NP // NTILE
B = 128
EP = 2560 * B
EDGES_PER_TILE = EP // NTILE
NCHUNK = EDGES_PER_TILE // B
GRP = 4
NGROUP = NCHUNK // GRP


def _make_sc_agg(compute_deg):
    mesh = plsc.VectorSubcoreMesh(core_axis_name="c", subcore_axis_name="s")
    out_type = [jax.ShapeDtypeStruct((NP, D), jnp.bfloat16),
                jax.ShapeDtypeStruct((NP, D), jnp.bfloat16)]
    if compute_deg:
        out_type += [jax.ShapeDtypeStruct((NP,), jnp.float32),
                     jax.ShapeDtypeStruct((NP,), jnp.float32)]
    scratch_types = (
        [pltpu.VMEM((2 * GRP, B), jnp.int32)] * 2 +
        [pltpu.VMEM((B, D), jnp.bfloat16)] * 2 +
        [pltpu.VMEM((B,), jnp.float32),
         pltpu.VMEM((ROWS_PER_TILE,), jnp.float32),
         pltpu.VMEM_SHARED((NP, D), jnp.bfloat16),
         pltpu.VMEM_SHARED((NP,), jnp.float32)] +
        [pltpu.SemaphoreType.DMA] * 8
    )

    @functools.partial(
        pl.kernel, mesh=mesh, out_type=out_type, scratch_types=scratch_types,
        compiler_params=pltpu.CompilerParams(use_tc_tiling_on_sc=False))
    def sc_agg(xa, xb, eda, edb, *refs):
        if compute_deg:
            agg_a, agg_b, deg_a, deg_b = refs[:4]
            rest = refs[4:]
        else:
            agg_a, agg_b = refs[:2]
            deg_a = deg_b = None
            rest = refs[2:]
        idx = list(rest[0:2])
        rows = list(rest[2:4])
        ones_v, zeros_v, agg_sh, deg_sh = rest[4:8]
        isem = list(rest[8:10])
        gsem = list(rest[10:12])
        ssem = list(rest[12:14])
        dsem = list(rest[14:16])
        rows0 = rows[0]

        c = lax.axis_index("c")
        s = lax.axis_index("s")
        row0 = s * ROWS_PER_TILE

        def zrow(j, _):
            r = j // (D // 32)
            k = j % (D // 32)
            rows0[r, pl.ds(k * 32, 32)] = jnp.zeros((32,), jnp.bfloat16)
            return 0
        lax.fori_loop(0, (B * D) // 32, zrow, 0)

        def zvec(j, _):
            zeros_v[pl.ds(j * 16, 16)] = jnp.zeros((16,), jnp.float32)
            return 0
        lax.fori_loop(0, ROWS_PER_TILE // 16, zvec, 0)
        if compute_deg:
            def ovec(j, _):
                ones_v[pl.ds(j * 16, 16)] = jnp.ones((16,), jnp.float32)
                return 0
            lax.fori_loop(0, B // 16, ovec, 0)

        for j in range(ROWS_PER_TILE // B):
            pltpu.sync_copy(rows0, agg_sh.at[pl.ds(row0 + j * B, B)])
        if compute_deg:
            pltpu.sync_copy(zeros_v, deg_sh.at[pl.ds(row0, ROWS_PER_TILE)])
        plsc.subcore_barrier()

        def run(x_hbm, ed_hbm):
            gbase = s * NGROUP

            def substep(g, sg, j):
                i = GRP * g + j
                cur, nxt = idx[sg], idx[1 - sg]
                p, q = j % 2, (j + 1) % 2
                prev_dst = cur.at[2 * j - 1] if j >= 1 else nxt.at[2 * GRP - 1]
                pltpu.make_async_copy(x_hbm.at[cur.at[2 * j]], rows[p],
                                      gsem[p]).wait()
                @pl.when(i + 1 < NCHUNK)
                def _():
                    @pl.when(i >= 1)
                    def _():
                        pltpu.make_async_copy(rows[q], agg_sh.at[prev_dst],
                                              ssem[q]).wait()
                        if compute_deg:
                            pltpu.make_async_copy(ones_v, deg_sh.at[prev_dst],
                                                  dsem[q]).wait()
                    if j == 0:
                        @pl.when((g >= 1) & (g + 1 < NGROUP))
                        def _():
                            pltpu.async_copy(ed_hbm.at[gbase + g + 1], nxt,
                                             isem[1 - sg])
                    if j < GRP - 1:
                        pltpu.async_copy(x_hbm.at[cur.at[2 * j + 2]], rows[q],
                                         gsem[q])
                    else:
                        pltpu.make_async_copy(ed_hbm.at[gbase + g + 1], nxt,
                                              isem[1 - sg]).wait()
                        pltpu.async_copy(x_hbm.at[nxt.at[0]], rows[q], gsem[q])
                pltpu.async_copy(rows[p], agg_sh.at[cur.at[2 * j + 1]], ssem[p],
                                 add=True)
                if compute_deg:
                    pltpu.async_copy(ones_v, deg_sh.at[cur.at[2 * j + 1]],
                                     dsem[p], add=True)

            pltpu.sync_copy(ed_hbm.at[gbase], idx[0])
            pltpu.async_copy(ed_hbm.at[gbase + 1], idx[1], isem[1])
            pltpu.async_copy(x_hbm.at[idx[0].at[0]], rows[0], gsem[0])

            def body(G, _):
                for gg in range(2):
                    for j in range(GRP):
                        substep(2 * G + gg, gg, j)
                return 0
            lax.fori_loop(0, NGROUP // 2, body, 0)
            lastbuf = idx[(NGROUP - 1) % 2]
            for j in (NCHUNK - 2, NCHUNK - 1):
                jj = j % GRP
                pltpu.make_async_copy(rows[j % 2],
                                      agg_sh.at[lastbuf.at[2 * jj + 1]],
                                      ssem[j % 2]).wait()
                if compute_deg:
                    pltpu.make_async_copy(ones_v,
                                          deg_sh.at[lastbuf.at[2 * jj + 1]],
                                          dsem[j % 2]).wait()

        @pl.when(c == 0)
        def _():
            run(xa, eda)

        @pl.when(c == 1)
        def _():
            run(xb, edb)

        plsc.subcore_barrier()

        def write_out(agg_out, deg_out):
            pltpu.sync_copy(agg_sh.at[pl.ds(row0, ROWS_PER_TILE)],
                            agg_out.at[pl.ds(row0, ROWS_PER_TILE)])
            if compute_deg:
                pltpu.sync_copy(deg_sh.at[pl.ds(row0, ROWS_PER_TILE)],
                                deg_out.at[pl.ds(row0, ROWS_PER_TILE)])

        @pl.when(c == 0)
        def _():
            write_out(agg_a, deg_a)

        @pl.when(c == 1)
        def _():
            write_out(agg_b, deg_b)

    return sc_agg


_sc_agg_deg = _make_sc_agg(compute_deg=True)
_sc_agg = _make_sc_agg(compute_deg=False)


BR = 1024


def _layer_body(x_ref, aa_ref, ab_ref, da_ref, db_ref,
                wsa_ref, wna_ref, wsb_ref, wnb_ref,
                oa_ref, ob_ref, oabf_ref, obbf_ref):
    xb = x_ref[...]
    ma = aa_ref[...].astype(jnp.float32) * (1.0 / jnp.maximum(da_ref[...], 1.0))
    mb = ab_ref[...].astype(jnp.float32) * (1.0 / jnp.maximum(db_ref[...], 1.0))
    ha = jnp.maximum(
        jnp.dot(xb, wsa_ref[...], preferred_element_type=jnp.float32)
        + jnp.dot(ma, wna_ref[...], preferred_element_type=jnp.float32), 0.0)
    hb = jnp.maximum(
        jnp.dot(xb, wsb_ref[...], preferred_element_type=jnp.float32)
        + jnp.dot(mb, wnb_ref[...], preferred_element_type=jnp.float32), 0.0)
    oa_ref[...] = ha
    ob_ref[...] = hb
    oabf_ref[...] = ha.astype(jnp.bfloat16)
    obbf_ref[...] = hb.astype(jnp.bfloat16)


def _tc_layer1(x, agg_a, agg_b, deg_a, deg_b, wsa, wna, wsb, wnb):
    row_spec = pl.BlockSpec((BR, D), lambda i: (i, 0))
    deg_spec = pl.BlockSpec((BR, 1), lambda i: (i, 0))
    w_spec = pl.BlockSpec((D, D), lambda i: (0, 0))
    return pl.pallas_call(
        _layer_body,
        grid=(NP // BR,),
        in_specs=[row_spec, row_spec, row_spec, deg_spec, deg_spec,
                  w_spec, w_spec, w_spec, w_spec],
        out_specs=[row_spec, row_spec, row_spec, row_spec],
        out_shape=[jax.ShapeDtypeStruct((NP, D), jnp.float32),
                   jax.ShapeDtypeStruct((NP, D), jnp.float32),
                   jax.ShapeDtypeStruct((NP, D), jnp.bfloat16),
                   jax.ShapeDtypeStruct((NP, D), jnp.bfloat16)],
    )(x, agg_a, agg_b, deg_a, deg_b, wsa, wna, wsb, wnb)


def _final_body(ha_ref, hb_ref, aa_ref, ab_ref, da_ref, db_ref,
                wsa_ref, wna_ref, wsb_ref, wnb_ref, wla_ref, wlb_ref,
                out_ref, acc_a, acc_b):
    i = pl.program_id(0)
    ma = aa_ref[...].astype(jnp.float32) * (1.0 / jnp.maximum(da_ref[...], 1.0))
    mb = ab_ref[...].astype(jnp.float32) * (1.0 / jnp.maximum(db_ref[...], 1.0))
    h2a = jnp.maximum(
        jnp.dot(ha_ref[...], wsa_ref[...], preferred_element_type=jnp.float32)
        + jnp.dot(ma, wna_ref[...], preferred_element_type=jnp.float32), 0.0)
    h2b = jnp.maximum(
        jnp.dot(hb_ref[...], wsb_ref[...], preferred_element_type=jnp.float32)
        + jnp.dot(mb, wnb_ref[...], preferred_element_type=jnp.float32), 0.0)
    rid = i * BR + lax.broadcasted_iota(jnp.int32, (BR, D), 0)
    mask = (rid < N).astype(jnp.float32)
    ca = jnp.sum(h2a * mask, axis=0, keepdims=True)
    cb = jnp.sum(h2b * mask, axis=0, keepdims=True)

    @pl.when(i == 0)
    def _():
        acc_a[...] = ca
        acc_b[...] = cb

    @pl.when(i > 0)
    def _():
        acc_a[...] += ca
        acc_b[...] += cb

    @pl.when(i == (NP // BR) - 1)
    def _():
        ra = jnp.dot(acc_a[...] * (1.0 / N), wla_ref[...],
                     preferred_element_type=jnp.float32)
        rb = jnp.dot(acc_b[...] * (1.0 / N), wlb_ref[...],
                     preferred_element_type=jnp.float32)
        sa = 1.0 / (1.0 + jnp.exp(-ra))
        sb = 1.0 / (1.0 + jnp.exp(-rb))
        out_ref[...] = (sa + sb) * 0.5


def _tc_final(h1a, h1b, agg_a, agg_b, deg_a, deg_b, wsa, wna, wsb, wnb, wla, wlb):
    row_spec = pl.BlockSpec((BR, D), lambda i: (i, 0))
    deg_spec = pl.BlockSpec((BR, 1), lambda i: (i, 0))
    w_spec = pl.BlockSpec((D, D), lambda i: (0, 0))
    wl_spec = pl.BlockSpec((D, C), lambda i: (0, 0))
    return pl.pallas_call(
        _final_body,
        grid=(NP // BR,),
        in_specs=[row_spec, row_spec, row_spec, row_spec, deg_spec, deg_spec,
                  w_spec, w_spec, w_spec, w_spec, wl_spec, wl_spec],
        out_specs=pl.BlockSpec((1, C), lambda i: (0, 0)),
        out_shape=jax.ShapeDtypeStruct((1, C), jnp.float32),
        scratch_shapes=[pltpu.VMEM((1, D), jnp.float32),
                        pltpu.VMEM((1, D), jnp.float32)],
    )(h1a, h1b, agg_a, agg_b, deg_a, deg_b, wsa, wna, wsb, wnb, wla, wlb)


def kernel(H1_edge_index, H2_edge_index, feats,
           W1s_a, W1n_a, W2s_a, W2n_a, Wl_a,
           W1s_b, W1n_b, W2s_b, W2n_b, Wl_b):
    pad_src = jnp.zeros((EP - E,), jnp.int32)
    pad_dst = jnp.full((EP - E,), N, jnp.int32)
    eda = jnp.stack(
        [jnp.concatenate([H1_edge_index[0], pad_src]).reshape(EP // B, B),
         jnp.concatenate([H1_edge_index[1], pad_dst]).reshape(EP // B, B)],
        axis=1).reshape(EP // (GRP * B), 2 * GRP, B)
    edb = jnp.stack(
        [jnp.concatenate([H2_edge_index[0], pad_src]).reshape(EP // B, B),
         jnp.concatenate([H2_edge_index[1], pad_dst]).reshape(EP // B, B)],
        axis=1).reshape(EP // (GRP * B), 2 * GRP, B)
    x = jnp.pad(feats, ((0, NP - N), (0, 0)))
    xbf = x.astype(jnp.bfloat16)

    agg1_a, agg1_b, deg_a, deg_b = _sc_agg_deg(xbf, xbf, eda, edb)
    deg_a = deg_a.reshape(NP, 1)
    deg_b = deg_b.reshape(NP, 1)
    h1a, h1b, h1abf, h1bbf = _tc_layer1(x, agg1_a, agg1_b, deg_a, deg_b,
                                        W1s_a, W1n_a, W1s_b, W1n_b)
    agg2_a, agg2_b = _sc_agg(h1abf, h1bbf, eda, edb)
    return _tc_final(h1a, h1b, agg2_a, agg2_b, deg_a, deg_b,
                     W2s_a, W2n_a, W2s_b, W2n_b, Wl_a, Wl_b)
